# baseline (speedup 1.0000x reference)
import functools

import jax
import jax.numpy as jnp
from jax import lax
from jax.experimental import pallas as pl
from jax.experimental.pallas import tpu as pltpu
from jax.experimental.pallas import tpu_sc as plsc

NC = 2
NS = 16
NW = NC * NS

NBUF = 7


def _kernel_impl(qus, table):
    batch, seq = qus.shape
    vocab, dim = table.shape
    assert batch % NW == 0
    b_per_w = batch // NW

    idx_t = qus.astype(jnp.int32).T

    mesh = plsc.VectorSubcoreMesh(core_axis_name="c", subcore_axis_name="s")
    LAG = NBUF - 1

    @functools.partial(
        pl.kernel,
        out_type=jax.ShapeDtypeStruct((seq, batch, dim), jnp.float32),
        mesh=mesh,
        scratch_types=[
            pltpu.VMEM((seq, b_per_w), jnp.int32),
            pltpu.VMEM((NBUF, b_per_w, dim), jnp.float32),
            [pltpu.SemaphoreType.DMA] * NBUF,
            [pltpu.SemaphoreType.DMA] * NBUF,
        ],
    )
    def emb(idx_hbm, table_hbm, out_hbm, idx_v, rows_v, gsems, wsems):
        wid = lax.axis_index("s") * NC + lax.axis_index("c")
        base_b = pl.multiple_of(wid * b_per_w, b_per_w)
        pltpu.sync_copy(idx_hbm.at[pl.ds(0, seq), pl.ds(base_b, b_per_w)], idx_v)
        gd = [None] * NBUF
        wd = [None] * NBUF
        for j in range(seq + LAG):
            if j < seq:
                b = j % NBUF
                if wd[b] is not None:
                    wd[b].wait()
                    wd[b] = None
                gd[b] = pltpu.async_copy(
                    table_hbm.at[idx_v.at[j]], rows_v.at[b], gsems[b]
                )
            k = j - LAG
            if k >= 0:
                bk = k % NBUF
                gd[bk].wait()
                wd[bk] = pltpu.async_copy(
                    rows_v.at[bk],
                    out_hbm.at[k, pl.ds(base_b, b_per_w)],
                    wsems[bk],
                )
        for b in range(NBUF):
            if wd[b] is not None:
                wd[b].wait()

    out_phys = emb(idx_t, table)
    return out_phys.transpose(1, 0, 2)


kernel = jax.jit(_kernel_impl)


# speedup vs baseline: 1.0096x; 1.0096x over previous
import functools

import jax
import jax.numpy as jnp
from jax import lax
from jax.experimental import pallas as pl
from jax.experimental.pallas import tpu as pltpu
from jax.experimental.pallas import tpu_sc as plsc

NC = 2
NS = 16
NW = NC * NS

NBUF = 6


def _kernel_impl(qus, table):
    batch, seq = qus.shape
    vocab, dim = table.shape
    assert batch % NW == 0
    b_per_w = batch // NW

    idx_t = qus.astype(jnp.int32).T

    mesh = plsc.VectorSubcoreMesh(core_axis_name="c", subcore_axis_name="s")
    LAG = NBUF - 1

    @functools.partial(
        pl.kernel,
        out_type=jax.ShapeDtypeStruct((seq, batch, dim), jnp.float32),
        mesh=mesh,
        scratch_types=[
            pltpu.VMEM((seq, b_per_w), jnp.int32),
            pltpu.VMEM((NBUF, b_per_w, dim), jnp.float32),
            [pltpu.SemaphoreType.DMA] * NBUF,
            [pltpu.SemaphoreType.DMA] * NBUF,
        ],
    )
    def emb(idx_hbm, table_hbm, out_hbm, idx_v, rows_v, gsems, wsems):
        wid = lax.axis_index("s") * NC + lax.axis_index("c")
        base_b = pl.multiple_of(wid * b_per_w, b_per_w)
        pltpu.sync_copy(idx_hbm.at[pl.ds(0, seq), pl.ds(base_b, b_per_w)], idx_v)
        gd = [None] * NBUF
        wd = [None] * NBUF
        for j in range(seq + LAG):
            if j < seq:
                b = j % NBUF
                if wd[b] is not None:
                    wd[b].wait()
                    wd[b] = None
                gd[b] = pltpu.async_copy(
                    table_hbm.at[idx_v.at[j]], rows_v.at[b], gsems[b]
                )
            k = j - LAG
            if k >= 0:
                bk = k % NBUF
                gd[bk].wait()
                wd[bk] = pltpu.async_copy(
                    rows_v.at[bk],
                    out_hbm.at[k, pl.ds(base_b, b_per_w)],
                    wsems[bk],
                )
        for b in range(NBUF):
            if wd[b] is not None:
                wd[b].wait()

    out_phys = emb(idx_t, table)
    return out_phys.transpose(1, 0, 2)


kernel = jax.jit(_kernel_impl)
